# baseline (device time: 19408 ns/iter reference)
import jax
import jax.numpy as jnp
from jax import lax
from jax.experimental import pallas as pl
from jax.experimental.pallas import tpu as pltpu

N_DEV = 16
N_Z = 4
N_P = 4


def kernel(x, router_W, route_idx, expert_W):
    m, d = x.shape
    e_per, _, h = expert_W.shape
    n_exp = router_W.shape[1]

    def body(x_ref, rw_ref, idx_ref, ew_ref, out_ref, buf_ref, sc_ref,
             up_ssem, up_rsem, dn_ssem, dn_rsem, b_ssem, b_rsem,
             u2_ssem, u2_rsem, d2_ssem, d2_rsem, c2_ssem, c2_rsem):
        my = lax.axis_index("i")
        my_z = my // N_P
        my_p = lax.rem(my, N_P)

        ewf = ew_ref[...]
        s_e = jnp.max(jnp.abs(ewf), axis=(1, 2), keepdims=True)
        q8 = jnp.clip(jnp.round(ewf * (127.0 / s_e)), -127.0, 127.0)
        buf_ref[0, 0] = q8.astype(jnp.int8)
        sc_ref[0, 0] = (s_e * (1.0 / 127.0)).reshape(e_per)

        barrier = pltpu.get_barrier_semaphore()
        for q in range(1, N_P):
            peer = my_z * N_P + lax.rem(my_p + q, N_P)
            pl.semaphore_signal(
                barrier, inc=1,
                device_id=(peer,), device_id_type=pl.DeviceIdType.MESH,
            )
        pl.when(my_z < N_Z - 1)(lambda: pl.semaphore_signal(
            barrier, inc=1,
            device_id=(my + N_P,), device_id_type=pl.DeviceIdType.MESH,
        ))
        pl.when(my_z > 0)(lambda: pl.semaphore_signal(
            barrier, inc=1,
            device_id=(my - N_P,), device_id_type=pl.DeviceIdType.MESH,
        ))
        n_nbrs = (N_P - 1) + (my_z < N_Z - 1).astype(jnp.int32) \
            + (my_z > 0).astype(jnp.int32)
        pl.semaphore_wait(barrier, n_nbrs)

        def up_send(dz, ref, ssem, rsem):
            return pltpu.make_async_remote_copy(
                src_ref=ref.at[0, 0],
                dst_ref=ref.at[N_Z - dz, 0],
                send_sem=ssem.at[dz - 1],
                recv_sem=rsem.at[dz - 1],
                device_id=(my + dz * N_P,),
                device_id_type=pl.DeviceIdType.MESH,
            )

        def dn_send(dz, ref, ssem, rsem):
            return pltpu.make_async_remote_copy(
                src_ref=ref.at[0, 0],
                dst_ref=ref.at[dz, 0],
                send_sem=ssem.at[dz - 1],
                recv_sem=rsem.at[dz - 1],
                device_id=(my - dz * N_P,),
                device_id_type=pl.DeviceIdType.MESH,
            )

        def plane_send(q, k):
            return pltpu.make_async_remote_copy(
                src_ref=buf_ref.at[k, 0],
                dst_ref=buf_ref.at[k, N_P - q],
                send_sem=b_ssem.at[q, k],
                recv_sem=b_rsem.at[N_P - q, k],
                device_id=(my_z * N_P + lax.rem(my_p + q, N_P),),
                device_id_type=pl.DeviceIdType.MESH,
            )

        def plane_send_sc(q, k):
            return pltpu.make_async_remote_copy(
                src_ref=sc_ref.at[k, 0],
                dst_ref=sc_ref.at[k, N_P - q],
                send_sem=c2_ssem.at[q, k],
                recv_sem=c2_rsem.at[N_P - q, k],
                device_id=(my_z * N_P + lax.rem(my_p + q, N_P),),
                device_id_type=pl.DeviceIdType.MESH,
            )

        def relay(k):
            for q in range(1, N_P):
                plane_send(q, k).start()
                plane_send_sc(q, k).start()

        for dz in range(1, N_Z):
            up_ok = my_z + dz <= N_Z - 1
            dn_ok = my_z - dz >= 0
            pl.when(up_ok)(lambda dz=dz: up_send(dz, buf_ref, up_ssem, up_rsem).start())
            pl.when(up_ok)(lambda dz=dz: up_send(dz, sc_ref, u2_ssem, u2_rsem).start())
            pl.when(dn_ok)(lambda dz=dz: dn_send(dz, buf_ref, dn_ssem, dn_rsem).start())
            pl.when(dn_ok)(lambda dz=dz: dn_send(dz, sc_ref, d2_ssem, d2_rsem).start())
        relay(0)

        for dz in range(1, N_Z):
            fb_pred = my_z >= dz
            fa_pred = my_z <= N_Z - 1 - dz
            pl.when(fb_pred)(lambda dz=dz: up_send(dz, buf_ref, up_ssem, up_rsem).wait_recv())
            pl.when(fb_pred)(lambda dz=dz: up_send(dz, sc_ref, u2_ssem, u2_rsem).wait_recv())
            pl.when(fb_pred)(lambda dz=dz: relay(N_Z - dz))
            pl.when(fa_pred)(lambda dz=dz: dn_send(dz, buf_ref, dn_ssem, dn_rsem).wait_recv())
            pl.when(fa_pred)(lambda dz=dz: dn_send(dz, sc_ref, d2_ssem, d2_rsem).wait_recv())
            pl.when(fa_pred)(lambda dz=dz: relay(dz))

        xf = x_ref[...]
        scores = jnp.dot(xf, rw_ref[...], preferred_element_type=jnp.float32)
        s_max = jnp.max(scores, axis=-1, keepdims=True)
        probs = jnp.exp(scores - s_max)
        probs = probs / jnp.sum(probs, axis=-1, keepdims=True)

        idx = idx_ref[...]
        idx0, idx1 = idx[:, 0:1], idx[:, 1:2]
        eids = lax.broadcasted_iota(jnp.int32, (m, n_exp), 1)
        g0 = jnp.sum(jnp.where(eids == idx0, probs, 0.0), axis=-1, keepdims=True)
        g1 = jnp.sum(jnp.where(eids == idx1, probs, 0.0), axis=-1, keepdims=True)
        gs = g0 + g1
        g0, g1 = g0 / gs, g1 / gs

        kk = eids // (N_P * e_per)
        jj = lax.rem(eids // e_per, N_P)
        ee = lax.rem(eids, e_per)
        slot_eids = (lax.rem(my_z + kk, N_Z) * N_P
                     + lax.rem(my_p + jj, N_P)) * e_per + ee
        g_slot = (jnp.where(slot_eids == idx0, g0, 0.0)
                  + jnp.where(slot_eids == idx1, g1, 0.0))

        a3 = (g_slot[:, :, None] * xf[:, None, :]).astype(jnp.bfloat16)

        bcol = lax.broadcasted_iota(jnp.int32, (1, e_per, 1), 1)
        def contrib(k, j):
            f_kj = jnp.zeros((1, e_per, 1), jnp.float32)
            for e in range(e_per):
                f_kj = jnp.where(bcol == e, sc_ref[k, j, e], f_kj)
            c = (k * N_P + j) * e_per
            a_kj = (a3[:, c:c + e_per, :]
                    * f_kj.astype(jnp.bfloat16)).reshape(m, e_per * d)
            w_kj = buf_ref[k, j].reshape(e_per * d, h).astype(jnp.bfloat16)
            return jnp.dot(a_kj, w_kj, preferred_element_type=jnp.float32)

        acc = contrib(0, 0)
        for k in range(1, N_Z):
            acc = acc + contrib(k, 0)

        for k in (0, 1, 3, 2):
            for j in (1, 3, 2):
                for ref, wsem, rsem in ((buf_ref, b_ssem, b_rsem),
                                        (sc_ref, c2_ssem, c2_rsem)):
                    recv = pltpu.make_async_remote_copy(
                        src_ref=ref.at[k, j],
                        dst_ref=ref.at[k, j],
                        send_sem=wsem.at[j, k],
                        recv_sem=rsem.at[j, k],
                        device_id=(my,),
                        device_id_type=pl.DeviceIdType.MESH,
                    )
                    recv.wait_recv()
                acc = acc + contrib(k, j)
        out_ref[...] = acc

        for dz in range(1, N_Z):
            up_ok = my_z + dz <= N_Z - 1
            dn_ok = my_z - dz >= 0
            pl.when(up_ok)(lambda dz=dz: up_send(dz, buf_ref, up_ssem, up_rsem).wait_send())
            pl.when(up_ok)(lambda dz=dz: up_send(dz, sc_ref, u2_ssem, u2_rsem).wait_send())
            pl.when(dn_ok)(lambda dz=dz: dn_send(dz, buf_ref, dn_ssem, dn_rsem).wait_send())
            pl.when(dn_ok)(lambda dz=dz: dn_send(dz, sc_ref, d2_ssem, d2_rsem).wait_send())
        for q in range(1, N_P):
            plane_send(q, 0).wait_send()
            plane_send_sc(q, 0).wait_send()
        for dz in range(1, N_Z):
            fb_pred = my_z >= dz
            fa_pred = my_z <= N_Z - 1 - dz
            for q in range(1, N_P):
                pl.when(fb_pred)(lambda dz=dz, q=q: plane_send(q, N_Z - dz).wait_send())
                pl.when(fb_pred)(lambda dz=dz, q=q: plane_send_sc(q, N_Z - dz).wait_send())
                pl.when(fa_pred)(lambda dz=dz, q=q: plane_send(q, dz).wait_send())
                pl.when(fa_pred)(lambda dz=dz, q=q: plane_send_sc(q, dz).wait_send())

    return pl.pallas_call(
        body,
        out_shape=jax.ShapeDtypeStruct((m, h), jnp.float32),
        in_specs=[
            pl.BlockSpec(memory_space=pltpu.VMEM),
            pl.BlockSpec(memory_space=pltpu.VMEM),
            pl.BlockSpec(memory_space=pltpu.VMEM),
            pl.BlockSpec(memory_space=pltpu.VMEM),
        ],
        out_specs=pl.BlockSpec(memory_space=pltpu.VMEM),
        scratch_shapes=[
            pltpu.VMEM((N_Z, N_P, e_per, d, h), jnp.int8),
            pltpu.VMEM((N_Z, N_P, e_per), jnp.float32),
            pltpu.SemaphoreType.DMA((N_Z - 1,)),
            pltpu.SemaphoreType.DMA((N_Z - 1,)),
            pltpu.SemaphoreType.DMA((N_Z - 1,)),
            pltpu.SemaphoreType.DMA((N_Z - 1,)),
            pltpu.SemaphoreType.DMA((N_P, N_Z)),
            pltpu.SemaphoreType.DMA((N_P, N_Z)),
            pltpu.SemaphoreType.DMA((N_Z - 1,)),
            pltpu.SemaphoreType.DMA((N_Z - 1,)),
            pltpu.SemaphoreType.DMA((N_Z - 1,)),
            pltpu.SemaphoreType.DMA((N_Z - 1,)),
            pltpu.SemaphoreType.DMA((N_P, N_Z)),
            pltpu.SemaphoreType.DMA((N_P, N_Z)),
        ],
        compiler_params=pltpu.CompilerParams(collective_id=0),
    )(x, router_W, route_idx, expert_W)


# device time: 19247 ns/iter; 1.0084x vs baseline; 1.0084x over previous
import jax
import jax.numpy as jnp
from jax import lax
from jax.experimental import pallas as pl
from jax.experimental.pallas import tpu as pltpu

N_DEV = 16
N_Z = 4
N_P = 4


def kernel(x, router_W, route_idx, expert_W):
    m, d = x.shape
    e_per, _, h = expert_W.shape
    n_exp = router_W.shape[1]

    def body(x_ref, rw_ref, idx_ref, ew_ref, out_ref, buf_ref, sc_ref,
             up_ssem, up_rsem, dn_ssem, dn_rsem, b_ssem, b_rsem,
             u2_ssem, u2_rsem, d2_ssem, d2_rsem, c2_ssem, c2_rsem):
        my = lax.axis_index("i")
        my_z = my // N_P
        my_p = lax.rem(my, N_P)

        ewf = ew_ref[...]
        s_e = jnp.max(jnp.abs(ewf), axis=(1, 2), keepdims=True)
        q8 = jnp.clip(jnp.round(ewf * (127.0 / s_e)), -127.0, 127.0)
        buf_ref[0, 0] = q8.astype(jnp.int8)
        sc_ref[0, 0] = (s_e * (1.0 / 127.0)).reshape(e_per)

        barrier = pltpu.get_barrier_semaphore()
        for q in range(1, N_P):
            peer = my_z * N_P + lax.rem(my_p + q, N_P)
            pl.semaphore_signal(
                barrier, inc=1,
                device_id=(peer,), device_id_type=pl.DeviceIdType.MESH,
            )
        pl.when(my_z < N_Z - 1)(lambda: pl.semaphore_signal(
            barrier, inc=1,
            device_id=(my + N_P,), device_id_type=pl.DeviceIdType.MESH,
        ))
        pl.when(my_z > 0)(lambda: pl.semaphore_signal(
            barrier, inc=1,
            device_id=(my - N_P,), device_id_type=pl.DeviceIdType.MESH,
        ))
        n_nbrs = (N_P - 1) + (my_z < N_Z - 1).astype(jnp.int32) \
            + (my_z > 0).astype(jnp.int32)
        pl.semaphore_wait(barrier, n_nbrs)

        def up_send(dz, ref, ssem, rsem):
            return pltpu.make_async_remote_copy(
                src_ref=ref.at[0, 0],
                dst_ref=ref.at[N_Z - dz, 0],
                send_sem=ssem.at[dz - 1],
                recv_sem=rsem.at[dz - 1],
                device_id=(my + dz * N_P,),
                device_id_type=pl.DeviceIdType.MESH,
            )

        def dn_send(dz, ref, ssem, rsem):
            return pltpu.make_async_remote_copy(
                src_ref=ref.at[0, 0],
                dst_ref=ref.at[dz, 0],
                send_sem=ssem.at[dz - 1],
                recv_sem=rsem.at[dz - 1],
                device_id=(my - dz * N_P,),
                device_id_type=pl.DeviceIdType.MESH,
            )

        def plane_send(q, k):
            return pltpu.make_async_remote_copy(
                src_ref=buf_ref.at[k, 0],
                dst_ref=buf_ref.at[k, N_P - q],
                send_sem=b_ssem.at[q, k],
                recv_sem=b_rsem.at[N_P - q, k],
                device_id=(my_z * N_P + lax.rem(my_p + q, N_P),),
                device_id_type=pl.DeviceIdType.MESH,
            )

        def plane_send_sc(q, k):
            return pltpu.make_async_remote_copy(
                src_ref=sc_ref.at[k, 0],
                dst_ref=sc_ref.at[k, N_P - q],
                send_sem=c2_ssem.at[q, k],
                recv_sem=c2_rsem.at[N_P - q, k],
                device_id=(my_z * N_P + lax.rem(my_p + q, N_P),),
                device_id_type=pl.DeviceIdType.MESH,
            )

        def relay(k):
            for q in range(1, N_P):
                plane_send(q, k).start()
                plane_send_sc(q, k).start()

        for dz in range(1, N_Z):
            up_ok = my_z + dz <= N_Z - 1
            dn_ok = my_z - dz >= 0
            pl.when(up_ok)(lambda dz=dz: up_send(dz, buf_ref, up_ssem, up_rsem).start())
            pl.when(up_ok)(lambda dz=dz: up_send(dz, sc_ref, u2_ssem, u2_rsem).start())
            pl.when(dn_ok)(lambda dz=dz: dn_send(dz, buf_ref, dn_ssem, dn_rsem).start())
            pl.when(dn_ok)(lambda dz=dz: dn_send(dz, sc_ref, d2_ssem, d2_rsem).start())
        relay(0)

        for dz in range(1, N_Z):
            fb_pred = my_z >= dz
            fa_pred = my_z <= N_Z - 1 - dz
            pl.when(fb_pred)(lambda dz=dz: up_send(dz, buf_ref, up_ssem, up_rsem).wait_recv())
            pl.when(fb_pred)(lambda dz=dz: up_send(dz, sc_ref, u2_ssem, u2_rsem).wait_recv())
            pl.when(fb_pred)(lambda dz=dz: relay(N_Z - dz))
            pl.when(fa_pred)(lambda dz=dz: dn_send(dz, buf_ref, dn_ssem, dn_rsem).wait_recv())
            pl.when(fa_pred)(lambda dz=dz: dn_send(dz, sc_ref, d2_ssem, d2_rsem).wait_recv())
            pl.when(fa_pred)(lambda dz=dz: relay(dz))

        xf = x_ref[...]
        scores = jnp.dot(xf, rw_ref[...], preferred_element_type=jnp.float32)
        s_max = jnp.max(scores, axis=-1, keepdims=True)
        probs = jnp.exp(scores - s_max)
        probs = probs / jnp.sum(probs, axis=-1, keepdims=True)

        idx = idx_ref[...]
        idx0, idx1 = idx[:, 0:1], idx[:, 1:2]
        eids = lax.broadcasted_iota(jnp.int32, (m, n_exp), 1)
        g0 = jnp.sum(jnp.where(eids == idx0, probs, 0.0), axis=-1, keepdims=True)
        g1 = jnp.sum(jnp.where(eids == idx1, probs, 0.0), axis=-1, keepdims=True)
        gs = g0 + g1
        g0, g1 = g0 / gs, g1 / gs

        kk = eids // (N_P * e_per)
        jj = lax.rem(eids // e_per, N_P)
        ee = lax.rem(eids, e_per)
        slot_eids = (lax.rem(my_z + kk, N_Z) * N_P
                     + lax.rem(my_p + jj, N_P)) * e_per + ee
        g_slot = (jnp.where(slot_eids == idx0, g0, 0.0)
                  + jnp.where(slot_eids == idx1, g1, 0.0))

        a3 = (g_slot[:, :, None] * xf[:, None, :]).astype(jnp.bfloat16)

        blk = N_P * e_per
        bcol = lax.broadcasted_iota(jnp.int32, (1, blk, 1), 1)
        acc = None
        for k in (0, 1, 3, 2):
            for j in (1, 3, 2):
                for ref, wsem, rsem in ((buf_ref, b_ssem, b_rsem),
                                        (sc_ref, c2_ssem, c2_rsem)):
                    recv = pltpu.make_async_remote_copy(
                        src_ref=ref.at[k, j],
                        dst_ref=ref.at[k, j],
                        send_sem=wsem.at[j, k],
                        recv_sem=rsem.at[j, k],
                        device_id=(my,),
                        device_id_type=pl.DeviceIdType.MESH,
                    )
                    recv.wait_recv()
            f_k = jnp.zeros((1, blk, 1), jnp.float32)
            for j in range(N_P):
                for e in range(e_per):
                    f_k = jnp.where(bcol == j * e_per + e,
                                    sc_ref[k, j, e], f_k)
            a_k = (a3[:, k * blk:(k + 1) * blk, :]
                   * f_k.astype(jnp.bfloat16)).reshape(m, blk * d)
            w_k = buf_ref[k].reshape(N_P * e_per * d, h).astype(jnp.bfloat16)
            part = jnp.dot(a_k, w_k, preferred_element_type=jnp.float32)
            acc = part if acc is None else acc + part
        out_ref[...] = acc

        for dz in range(1, N_Z):
            up_ok = my_z + dz <= N_Z - 1
            dn_ok = my_z - dz >= 0
            pl.when(up_ok)(lambda dz=dz: up_send(dz, buf_ref, up_ssem, up_rsem).wait_send())
            pl.when(up_ok)(lambda dz=dz: up_send(dz, sc_ref, u2_ssem, u2_rsem).wait_send())
            pl.when(dn_ok)(lambda dz=dz: dn_send(dz, buf_ref, dn_ssem, dn_rsem).wait_send())
            pl.when(dn_ok)(lambda dz=dz: dn_send(dz, sc_ref, d2_ssem, d2_rsem).wait_send())
        for q in range(1, N_P):
            plane_send(q, 0).wait_send()
            plane_send_sc(q, 0).wait_send()
        for dz in range(1, N_Z):
            fb_pred = my_z >= dz
            fa_pred = my_z <= N_Z - 1 - dz
            for q in range(1, N_P):
                pl.when(fb_pred)(lambda dz=dz, q=q: plane_send(q, N_Z - dz).wait_send())
                pl.when(fb_pred)(lambda dz=dz, q=q: plane_send_sc(q, N_Z - dz).wait_send())
                pl.when(fa_pred)(lambda dz=dz, q=q: plane_send(q, dz).wait_send())
                pl.when(fa_pred)(lambda dz=dz, q=q: plane_send_sc(q, dz).wait_send())

    return pl.pallas_call(
        body,
        out_shape=jax.ShapeDtypeStruct((m, h), jnp.float32),
        in_specs=[
            pl.BlockSpec(memory_space=pltpu.VMEM),
            pl.BlockSpec(memory_space=pltpu.VMEM),
            pl.BlockSpec(memory_space=pltpu.VMEM),
            pl.BlockSpec(memory_space=pltpu.VMEM),
        ],
        out_specs=pl.BlockSpec(memory_space=pltpu.VMEM),
        scratch_shapes=[
            pltpu.VMEM((N_Z, N_P, e_per, d, h), jnp.int8),
            pltpu.VMEM((N_Z, N_P, e_per), jnp.float32),
            pltpu.SemaphoreType.DMA((N_Z - 1,)),
            pltpu.SemaphoreType.DMA((N_Z - 1,)),
            pltpu.SemaphoreType.DMA((N_Z - 1,)),
            pltpu.SemaphoreType.DMA((N_Z - 1,)),
            pltpu.SemaphoreType.DMA((N_P, N_Z)),
            pltpu.SemaphoreType.DMA((N_P, N_Z)),
            pltpu.SemaphoreType.DMA((N_Z - 1,)),
            pltpu.SemaphoreType.DMA((N_Z - 1,)),
            pltpu.SemaphoreType.DMA((N_Z - 1,)),
            pltpu.SemaphoreType.DMA((N_Z - 1,)),
            pltpu.SemaphoreType.DMA((N_P, N_Z)),
            pltpu.SemaphoreType.DMA((N_P, N_Z)),
        ],
        compiler_params=pltpu.CompilerParams(collective_id=0),
    )(x, router_W, route_idx, expert_W)
